# baseline (speedup 1.0000x reference)
import functools

import jax
import jax.numpy as jnp
import numpy as np
from jax import lax
from jax.experimental import pallas as pl
from jax.experimental.pallas import tpu as pltpu
from jax.experimental.pallas import tpu_sc as plsc

_B = 1024
_D = 64
_M = 100000
_H = 4
_K = 5
_T = 2048
_NT = (_M + _T - 1) // _T
_W = 32
_GPT = _T // _W
_NG = _M // _W
_NC = _K * _W
_BIG = 2 ** 30


def _gelu_exact(h):
    return 0.5 * h * (1.0 + lax.erf(h * np.float32(0.7071067811865476)))


def _enc_body(x_ref, w1_ref, b1_ref, g_ref, bt_ref, w2_ref, b2_ref,
              eq_ref, qn_ref):
    x = x_ref[...]
    h = jax.lax.dot_general(x, w1_ref[...], (((1,), (1,)), ((), ())),
                            preferred_element_type=jnp.float32) + b1_ref[...]
    mu = jnp.mean(h, axis=-1, keepdims=True)
    var = jnp.mean((h - mu) ** 2, axis=-1, keepdims=True)
    h = (h - mu) / jnp.sqrt(var + 1e-5) * g_ref[...] + bt_ref[...]
    h = _gelu_exact(h)
    eq = jax.lax.dot_general(h, w2_ref[...], (((1,), (1,)), ((), ())),
                             preferred_element_type=jnp.float32) + b2_ref[...]
    eq_ref[...] = eq
    nrm = jnp.sqrt(jnp.sum(eq * eq, axis=-1, keepdims=True))
    qn_ref[...] = eq / jnp.maximum(nrm, 1e-8)


def _encode(x, w1, b1, g, bt, w2, b2):
    return pl.pallas_call(
        _enc_body,
        out_shape=[jax.ShapeDtypeStruct((_B, _D), jnp.float32),
                   jax.ShapeDtypeStruct((_B, _D), jnp.float32)],
    )(x, w1, b1.reshape(1, _D), g.reshape(1, _D), bt.reshape(1, _D),
      w2, b2.reshape(1, _D))


def _knn_body(qn_ref, s_ref, cand_ref, cv_ref, ci_ref):
    i = pl.program_id(0)

    @pl.when(i == 0)
    def _init():
        cv_ref[...] = jnp.full((_B, 8), -jnp.inf, jnp.float32)
        ci_ref[...] = jnp.zeros((_B, 8), jnp.int32)

    s = s_ref[...]
    nrm = jnp.sqrt(jnp.sum(s * s, axis=1, keepdims=True))
    sn = s / jnp.maximum(nrm, 1e-8)
    sc = jax.lax.dot_general(qn_ref[...], sn, (((1,), (1,)), ((), ())),
                             preferred_element_type=jnp.float32)
    g = jnp.max(sc.reshape(_B, _GPT, _W), axis=2)
    lane_g = jax.lax.broadcasted_iota(jnp.int32, (1, _GPT), 1)
    gid = i * _GPT + lane_g
    g = jnp.where(gid < _NG, g, -jnp.inf)

    A = jnp.concatenate([cv_ref[...], g], axis=1)
    AI = jnp.concatenate(
        [ci_ref[...], jnp.broadcast_to(gid, (_B, _GPT))], axis=1)

    nv, ni = [], []
    for _ in range(_K):
        m = jnp.max(A, axis=1, keepdims=True)
        sel = jnp.min(jnp.where(A == m, AI, _BIG), axis=1, keepdims=True)
        nv.append(m)
        ni.append(sel)
        A = jnp.where(AI == sel, -jnp.inf, A)

    lane8 = jax.lax.broadcasted_iota(jnp.int32, (1, 8), 1)
    cv = jnp.full((_B, 8), -jnp.inf, jnp.float32)
    ci = jnp.zeros((_B, 8), jnp.int32)
    for j in range(_K):
        selj = lane8 == j
        cv = jnp.where(selj, nv[j], cv)
        ci = jnp.where(selj, ni[j], ci)
    cv_ref[...] = cv
    ci_ref[...] = ci

    @pl.when(i == _NT - 1)
    def _emit():
        lane_c = jax.lax.broadcasted_iota(jnp.int32, (1, _NC), 1)
        jsel = lane_c // _W
        grp = jnp.zeros((_B, _NC), jnp.int32)
        for j in range(_K):
            grp = jnp.where(jsel == j, ni[j], grp)
        cand_ref[...] = grp * _W + lane_c % _W


def _knn_groups(qn, storage):
    return pl.pallas_call(
        _knn_body,
        grid=(_NT,),
        in_specs=[
            pl.BlockSpec((_B, _D), lambda i: (0, 0)),
            pl.BlockSpec((_T, _D), lambda i: (i, 0)),
        ],
        out_specs=pl.BlockSpec((_B, _NC), lambda i: (0, 0)),
        out_shape=jax.ShapeDtypeStruct((_B, _NC), jnp.int32),
        scratch_shapes=[
            pltpu.VMEM((_B, 8), jnp.float32),
            pltpu.VMEM((_B, 8), jnp.int32),
        ],
        compiler_params=pltpu.CompilerParams(
            dimension_semantics=("arbitrary",)),
    )(qn, storage)


def _gather_rows(table, idx_flat, chunk):
    n = idx_flat.shape[0]
    nw = 32
    bpw = n // nw
    nch = bpw // chunk
    mesh = plsc.VectorSubcoreMesh(core_axis_name="c", subcore_axis_name="s")

    @functools.partial(
        pl.kernel,
        mesh=mesh,
        out_type=jax.ShapeDtypeStruct((n, _D), jnp.float32),
        scratch_types=[
            pltpu.VMEM((chunk,), jnp.int32),
            pltpu.VMEM((chunk, _D), jnp.float32),
            pltpu.SemaphoreType.DMA,
        ],
        compiler_params=pltpu.CompilerParams(use_tc_tiling_on_sc=False),
    )
    def k(table_hbm, idx_hbm, out_hbm, idx_v, rows_v, sem):
        wid = lax.axis_index("s") * 2 + lax.axis_index("c")
        base = wid * bpw
        for ch in range(nch):
            off = base + ch * chunk
            pltpu.sync_copy(idx_hbm.at[pl.ds(off, chunk)], idx_v)
            pltpu.async_copy(table_hbm.at[idx_v], rows_v, sem).wait()
            pltpu.sync_copy(rows_v, out_hbm.at[pl.ds(off, chunk)])

    return k(table, idx_flat)


def _resolve_body(qn_ref, rows_ref, cand_ref, oi_ref):
    qn = qn_ref[...]
    scs = []
    for c in range(_NC):
        seg = rows_ref[:, c * _D:(c + 1) * _D]
        d = jnp.sum(seg * qn, axis=1, keepdims=True)
        n2 = jnp.sum(seg * seg, axis=1, keepdims=True)
        scs.append(d / jnp.maximum(jnp.sqrt(n2), 1e-8))
    S = jnp.concatenate(scs, axis=1)
    AI = cand_ref[...]
    ids = []
    for _ in range(_K):
        m = jnp.max(S, axis=1, keepdims=True)
        sel = jnp.min(jnp.where(S == m, AI, _BIG), axis=1, keepdims=True)
        ids.append(sel)
        S = jnp.where(AI == sel, -jnp.inf, S)
    lane8 = jax.lax.broadcasted_iota(jnp.int32, (1, 8), 1)
    oi = jnp.zeros((qn.shape[0], 8), jnp.int32)
    for j in range(_K):
        oi = jnp.where(lane8 == j, ids[j], oi)
    oi_ref[...] = oi


def _resolve(qn, rows_flat, cand):
    nb = 4
    bb = _B // nb
    return pl.pallas_call(
        _resolve_body,
        grid=(nb,),
        in_specs=[
            pl.BlockSpec((bb, _D), lambda i: (i, 0)),
            pl.BlockSpec((bb, _NC * _D), lambda i: (i, 0)),
            pl.BlockSpec((bb, _NC), lambda i: (i, 0)),
        ],
        out_specs=pl.BlockSpec((bb, 8), lambda i: (i, 0)),
        out_shape=jax.ShapeDtypeStruct((_B, 8), jnp.int32),
        compiler_params=pltpu.CompilerParams(
            dimension_semantics=("parallel",)),
    )(qn, rows_flat, cand)


def _post_body(x_ref, eq_ref, r_ref, wq_ref, bq_ref, wk_ref, bk_ref,
               wv_ref, bv_ref, wo_ref, bo_ref, c1w_ref, c1b_ref,
               c2w_ref, c2b_ref, seg_ref, out_ref):
    eq = eq_ref[...]
    q = jax.lax.dot_general(eq, wq_ref[...], (((1,), (1,)), ((), ())),
                            preferred_element_type=jnp.float32) + bq_ref[...]
    seg = seg_ref[...]
    scs, vs = [], []
    for j in range(_K):
        r = r_ref[:, j * _D:(j + 1) * _D]
        kj = jax.lax.dot_general(r, wk_ref[...], (((1,), (1,)), ((), ())),
                                 preferred_element_type=jnp.float32) + bk_ref[...]
        vj = jax.lax.dot_general(r, wv_ref[...], (((1,), (1,)), ((), ())),
                                 preferred_element_type=jnp.float32) + bv_ref[...]
        sj = jax.lax.dot_general(q * kj, seg, (((1,), (1,)), ((), ())),
                                 preferred_element_type=jnp.float32)
        scs.append(sj * 0.25)
        vs.append(vj)
    m = scs[0]
    for j in range(1, _K):
        m = jnp.maximum(m, scs[j])
    es = [jnp.exp(s - m) for s in scs]
    tot = es[0]
    for j in range(1, _K):
        tot = tot + es[j]
    ctx = jnp.zeros((_B, _D), jnp.float32)
    for j in range(_K):
        a = es[j] / tot
        ab = jax.lax.dot_general(a, seg, (((1,), (0,)), ((), ())),
                                 preferred_element_type=jnp.float32)
        ctx = ctx + ab * vs[j]
    comp = jax.lax.dot_general(ctx, wo_ref[...], (((1,), (1,)), ((), ())),
                               preferred_element_type=jnp.float32) + bo_ref[...]
    h = jax.lax.dot_general(comp, c1w_ref[...], (((1,), (1,)), ((), ())),
                            preferred_element_type=jnp.float32) + c1b_ref[...]
    h = _gelu_exact(h)
    ca1 = jax.lax.dot_general(h, c2w_ref[...], (((1,), (1,)), ((), ())),
                              preferred_element_type=jnp.float32) + c2b_ref[...]
    out_ref[...] = x_ref[...] + 0.5 * ca1


def _post(x, eq, retrieved_flat, in_proj_w, in_proj_b, out_proj_w, out_proj_b,
          c1_W, c1_b, c2_W, c2_b):
    seg = np.zeros((_H, _D), np.float32)
    for h in range(_H):
        seg[h, h * 16:(h + 1) * 16] = 1.0
    seg = jnp.asarray(seg)
    wq, wk, wv = in_proj_w[:_D], in_proj_w[_D:2 * _D], in_proj_w[2 * _D:]
    bq, bk, bv = in_proj_b[:_D], in_proj_b[_D:2 * _D], in_proj_b[2 * _D:]
    return pl.pallas_call(
        _post_body,
        out_shape=jax.ShapeDtypeStruct((_B, _D), jnp.float32),
    )(x, eq, retrieved_flat, wq, bq.reshape(1, _D), wk, bk.reshape(1, _D),
      wv, bv.reshape(1, _D), out_proj_w, out_proj_b.reshape(1, _D),
      c1_W, c1_b.reshape(1, 2 * _D), c2_W, c2_b.reshape(1, _D), seg)


def kernel(x, k_W1, k_b1, k_gamma, k_beta, k_W2, k_b2, storage, memory_values,
           in_proj_w, in_proj_b, out_proj_w, out_proj_b, c1_W, c1_b, c2_W,
           c2_b):
    eq, qn = _encode(x, k_W1, k_b1, k_gamma, k_beta, k_W2, k_b2)
    cand = _knn_groups(qn, storage)
    crows = _gather_rows(storage, cand.reshape(_B * _NC), chunk=1024)
    top = _resolve(qn, crows.reshape(_B, _NC * _D), cand)
    idx_flat = top[:, :_K].reshape(_B * _K)
    retrieved = _gather_rows(memory_values, idx_flat, chunk=160)
    retrieved_flat = retrieved.reshape(_B, _K * _D)
    return _post(x, eq, retrieved_flat, in_proj_w, in_proj_b,
                 out_proj_w, out_proj_b, c1_W, c1_b, c2_W, c2_b)


# speedup vs baseline: 3.6552x; 3.6552x over previous
import functools

import jax
import jax.numpy as jnp
import numpy as np
from jax import lax
from jax.experimental import pallas as pl
from jax.experimental.pallas import tpu as pltpu
from jax.experimental.pallas import tpu_sc as plsc

_B = 1024
_D = 64
_M = 100000
_H = 4
_K = 5
_T = 2048
_NT = (_M + _T - 1) // _T
_MP = _NT * _T
_W = 16
_GPT = _T // _W
_NG = _M // _W
_NC = _K * _W
_BIG = 2 ** 30


def _gelu_exact(h):
    return 0.5 * h * (1.0 + lax.erf(h * np.float32(0.7071067811865476)))


def _enc_body(x_ref, w1_ref, b1_ref, g_ref, bt_ref, w2_ref, b2_ref,
              eq_ref, qn_ref):
    x = x_ref[...]
    h = jax.lax.dot_general(x, w1_ref[...], (((1,), (1,)), ((), ())),
                            preferred_element_type=jnp.float32) + b1_ref[...]
    mu = jnp.mean(h, axis=-1, keepdims=True)
    var = jnp.mean((h - mu) ** 2, axis=-1, keepdims=True)
    h = (h - mu) / jnp.sqrt(var + 1e-5) * g_ref[...] + bt_ref[...]
    h = _gelu_exact(h)
    eq = jax.lax.dot_general(h, w2_ref[...], (((1,), (1,)), ((), ())),
                             preferred_element_type=jnp.float32) + b2_ref[...]
    eq_ref[...] = eq
    nrm = jnp.sqrt(jnp.sum(eq * eq, axis=-1, keepdims=True))
    qn_ref[...] = eq / jnp.maximum(nrm, 1e-8)


def _encode(x, w1, b1, g, bt, w2, b2):
    return pl.pallas_call(
        _enc_body,
        out_shape=[jax.ShapeDtypeStruct((_B, _D), jnp.float32),
                   jax.ShapeDtypeStruct((_B, _D), jnp.float32)],
    )(x, w1, b1.reshape(1, _D), g.reshape(1, _D), bt.reshape(1, _D),
      w2, b2.reshape(1, _D))


def _knn_body(qn_ref, s_ref, cand_ref, sn_ref, cv_ref, ci_ref):
    i = pl.program_id(0)

    @pl.when(i == 0)
    def _init():
        cv_ref[...] = jnp.full((8, _B), -jnp.inf, jnp.float32)
        ci_ref[...] = jnp.zeros((8, _B), jnp.int32)

    s = s_ref[...]
    nrm = jnp.sqrt(jnp.sum(s * s, axis=1, keepdims=True))
    sn = s / jnp.maximum(nrm, 1e-8)
    sn_ref[...] = sn
    sc = jax.lax.dot_general(sn, qn_ref[...], (((1,), (1,)), ((), ())),
                             preferred_element_type=jnp.float32)
    g = jnp.max(sc.reshape(_GPT, _W, _B), axis=1)
    sub_g = jax.lax.broadcasted_iota(jnp.int32, (_GPT, 1), 0)
    gid = i * _GPT + sub_g
    g = jnp.where(gid < _NG, g, -jnp.inf)

    A = jnp.concatenate([cv_ref[...], g], axis=0)
    AI = jnp.concatenate(
        [ci_ref[...], jnp.broadcast_to(gid, (_GPT, _B))], axis=0)

    nv, ni = [], []
    for _ in range(_K):
        m = jnp.max(A, axis=0, keepdims=True)
        sel = jnp.min(jnp.where(A == m, AI, _BIG), axis=0, keepdims=True)
        nv.append(m)
        ni.append(sel)
        A = jnp.where(AI == sel, -jnp.inf, A)

    sub8 = jax.lax.broadcasted_iota(jnp.int32, (8, 1), 0)
    cv = jnp.full((8, _B), -jnp.inf, jnp.float32)
    ci = jnp.zeros((8, _B), jnp.int32)
    for j in range(_K):
        selj = sub8 == j
        cv = jnp.where(selj, nv[j], cv)
        ci = jnp.where(selj, ni[j], ci)
    cv_ref[...] = cv
    ci_ref[...] = ci

    @pl.when(i == _NT - 1)
    def _emit():
        sub_c = jax.lax.broadcasted_iota(jnp.int32, (_NC, 1), 0)
        jsel = sub_c // _W
        grp = jnp.zeros((_NC, _B), jnp.int32)
        for j in range(_K):
            grp = jnp.where(jsel == j, ni[j], grp)
        cand_ref[...] = grp * _W + sub_c % _W


def _knn_groups(qn, storage):
    return pl.pallas_call(
        _knn_body,
        grid=(_NT,),
        in_specs=[
            pl.BlockSpec((_B, _D), lambda i: (0, 0)),
            pl.BlockSpec((_T, _D), lambda i: (i, 0)),
        ],
        out_specs=[
            pl.BlockSpec((_NC, _B), lambda i: (0, 0)),
            pl.BlockSpec((_T, _D), lambda i: (i, 0)),
        ],
        out_shape=[jax.ShapeDtypeStruct((_NC, _B), jnp.int32),
                   jax.ShapeDtypeStruct((_MP, _D), jnp.float32)],
        scratch_shapes=[
            pltpu.VMEM((8, _B), jnp.float32),
            pltpu.VMEM((8, _B), jnp.int32),
        ],
        compiler_params=pltpu.CompilerParams(
            dimension_semantics=("arbitrary",)),
    )(qn, storage)


def _gather_rows(table, idx_flat, chunk):
    n = idx_flat.shape[0]
    nw = 32
    bpw = n // nw
    nch = bpw // chunk
    mesh = plsc.VectorSubcoreMesh(core_axis_name="c", subcore_axis_name="s")

    @functools.partial(
        pl.kernel,
        mesh=mesh,
        out_type=jax.ShapeDtypeStruct((n, _D), jnp.float32),
        scratch_types=[
            pltpu.VMEM((chunk,), jnp.int32),
            pltpu.VMEM((chunk, _D), jnp.float32),
            pltpu.SemaphoreType.DMA,
        ],
        compiler_params=pltpu.CompilerParams(use_tc_tiling_on_sc=False),
    )
    def k(table_hbm, idx_hbm, out_hbm, idx_v, rows_v, sem):
        wid = lax.axis_index("s") * 2 + lax.axis_index("c")
        base = wid * bpw
        for ch in range(nch):
            off = base + ch * chunk
            pltpu.sync_copy(idx_hbm.at[pl.ds(off, chunk)], idx_v)
            pltpu.async_copy(table_hbm.at[idx_v], rows_v, sem).wait()
            pltpu.sync_copy(rows_v, out_hbm.at[pl.ds(off, chunk)])

    return k(table, idx_flat)


def _resolve_body(qn_ref, rows_ref, cand_ref, oi_ref):
    qn = qn_ref[...]
    scs = []
    for c in range(_NC):
        seg = rows_ref[c]
        scs.append(jnp.sum(seg * qn, axis=1, keepdims=True))
    S = jnp.concatenate(scs, axis=1)
    AI = cand_ref[...]
    ids = []
    for _ in range(_K):
        m = jnp.max(S, axis=1, keepdims=True)
        sel = jnp.min(jnp.where(S == m, AI, _BIG), axis=1, keepdims=True)
        ids.append(sel)
        S = jnp.where(AI == sel, -jnp.inf, S)
    lane8 = jax.lax.broadcasted_iota(jnp.int32, (1, 8), 1)
    oi = jnp.zeros((qn.shape[0], 8), jnp.int32)
    for j in range(_K):
        oi = jnp.where(lane8 == j, ids[j], oi)
    oi_ref[...] = oi


def _resolve(qn, rows, cand_q):
    nb = 4
    bb = _B // nb
    return pl.pallas_call(
        _resolve_body,
        grid=(nb,),
        in_specs=[
            pl.BlockSpec((bb, _D), lambda i: (i, 0)),
            pl.BlockSpec((_NC, bb, _D), lambda i: (0, i, 0)),
            pl.BlockSpec((bb, _NC), lambda i: (i, 0)),
        ],
        out_specs=pl.BlockSpec((bb, 8), lambda i: (i, 0)),
        out_shape=jax.ShapeDtypeStruct((_B, 8), jnp.int32),
        compiler_params=pltpu.CompilerParams(
            dimension_semantics=("parallel",)),
    )(qn, rows, cand_q)


def _post_body(x_ref, eq_ref, r_ref, wq_ref, bq_ref, wk_ref, bk_ref,
               wv_ref, bv_ref, wo_ref, bo_ref, c1w_ref, c1b_ref,
               c2w_ref, c2b_ref, seg_ref, out_ref):
    eq = eq_ref[...]
    q = jax.lax.dot_general(eq, wq_ref[...], (((1,), (1,)), ((), ())),
                            preferred_element_type=jnp.float32) + bq_ref[...]
    seg = seg_ref[...]
    scs, vs = [], []
    for j in range(_K):
        r = r_ref[:, j * _D:(j + 1) * _D]
        kj = jax.lax.dot_general(r, wk_ref[...], (((1,), (1,)), ((), ())),
                                 preferred_element_type=jnp.float32) + bk_ref[...]
        vj = jax.lax.dot_general(r, wv_ref[...], (((1,), (1,)), ((), ())),
                                 preferred_element_type=jnp.float32) + bv_ref[...]
        sj = jax.lax.dot_general(q * kj, seg, (((1,), (1,)), ((), ())),
                                 preferred_element_type=jnp.float32)
        scs.append(sj * 0.25)
        vs.append(vj)
    m = scs[0]
    for j in range(1, _K):
        m = jnp.maximum(m, scs[j])
    es = [jnp.exp(s - m) for s in scs]
    tot = es[0]
    for j in range(1, _K):
        tot = tot + es[j]
    ctx = jnp.zeros((_B, _D), jnp.float32)
    for j in range(_K):
        a = es[j] / tot
        ab = jax.lax.dot_general(a, seg, (((1,), (0,)), ((), ())),
                                 preferred_element_type=jnp.float32)
        ctx = ctx + ab * vs[j]
    comp = jax.lax.dot_general(ctx, wo_ref[...], (((1,), (1,)), ((), ())),
                               preferred_element_type=jnp.float32) + bo_ref[...]
    h = jax.lax.dot_general(comp, c1w_ref[...], (((1,), (1,)), ((), ())),
                            preferred_element_type=jnp.float32) + c1b_ref[...]
    h = _gelu_exact(h)
    ca1 = jax.lax.dot_general(h, c2w_ref[...], (((1,), (1,)), ((), ())),
                              preferred_element_type=jnp.float32) + c2b_ref[...]
    out_ref[...] = x_ref[...] + 0.5 * ca1


def _post(x, eq, retrieved_flat, in_proj_w, in_proj_b, out_proj_w, out_proj_b,
          c1_W, c1_b, c2_W, c2_b):
    seg = np.zeros((_H, _D), np.float32)
    for h in range(_H):
        seg[h, h * 16:(h + 1) * 16] = 1.0
    seg = jnp.asarray(seg)
    wq, wk, wv = in_proj_w[:_D], in_proj_w[_D:2 * _D], in_proj_w[2 * _D:]
    bq, bk, bv = in_proj_b[:_D], in_proj_b[_D:2 * _D], in_proj_b[2 * _D:]
    return pl.pallas_call(
        _post_body,
        out_shape=jax.ShapeDtypeStruct((_B, _D), jnp.float32),
    )(x, eq, retrieved_flat, wq, bq.reshape(1, _D), wk, bk.reshape(1, _D),
      wv, bv.reshape(1, _D), out_proj_w, out_proj_b.reshape(1, _D),
      c1_W, c1_b.reshape(1, 2 * _D), c2_W, c2_b.reshape(1, _D), seg)


def kernel(x, k_W1, k_b1, k_gamma, k_beta, k_W2, k_b2, storage, memory_values,
           in_proj_w, in_proj_b, out_proj_w, out_proj_b, c1_W, c1_b, c2_W,
           c2_b):
    eq, qn = _encode(x, k_W1, k_b1, k_gamma, k_beta, k_W2, k_b2)
    cand_t, sn_full = _knn_groups(qn, storage)
    crows = _gather_rows(sn_full, cand_t.reshape(_NC * _B), chunk=1280)
    top = _resolve(qn, crows.reshape(_NC, _B, _D),
                   jnp.transpose(cand_t))
    idx_flat = top[:, :_K].reshape(_B * _K)
    retrieved = _gather_rows(memory_values, idx_flat, chunk=160)
    retrieved_flat = retrieved.reshape(_B, _K * _D)
    return _post(x, eq, retrieved_flat, in_proj_w, in_proj_b,
                 out_proj_w, out_proj_b, c1_W, c1_b, c2_W, c2_b)


# speedup vs baseline: 4.4195x; 1.2091x over previous
import functools

import jax
import jax.numpy as jnp
import numpy as np
from jax import lax
from jax.experimental import pallas as pl
from jax.experimental.pallas import tpu as pltpu
from jax.experimental.pallas import tpu_sc as plsc

_B = 1024
_D = 64
_DP = 128
_M = 100000
_H = 4
_K = 5
_T = 2048
_NT = (_M + _T - 1) // _T
_MP = _NT * _T
_W = 16
_GPT = _T // _W
_NG = _M // _W
_NC = _K * _W
_BIG = 2 ** 30


def _gelu_exact(h):
    return 0.5 * h * (1.0 + lax.erf(h * np.float32(0.7071067811865476)))


def _mm(a, b, dims):
    return jax.lax.dot_general(a, b, (dims, ((), ())),
                               preferred_element_type=jnp.float32)


def _knn_body(x_ref, w1_ref, b1_ref, g_ref, bt_ref, w2_ref, b2_ref,
              s_ref, cand_ref, sn_ref, eq_ref, qn_out_ref, qn_ref,
              cv_ref, ci_ref):
    i = pl.program_id(0)

    @pl.when(i == 0)
    def _init():
        x = x_ref[...]
        h = _mm(x, w1_ref[...], ((1,), (1,))) + b1_ref[...]
        mu = jnp.mean(h, axis=-1, keepdims=True)
        var = jnp.mean((h - mu) ** 2, axis=-1, keepdims=True)
        h = (h - mu) / jnp.sqrt(var + 1e-5) * g_ref[...] + bt_ref[...]
        h = _gelu_exact(h)
        eq = _mm(h, w2_ref[...], ((1,), (1,))) + b2_ref[...]
        eq_ref[...] = eq
        qnrm = jnp.sqrt(jnp.sum(eq * eq, axis=-1, keepdims=True))
        qn = eq / jnp.maximum(qnrm, 1e-8)
        qn_ref[...] = qn
        qn_out_ref[...] = qn
        cv_ref[...] = jnp.full((8, _B), -jnp.inf, jnp.float32)
        ci_ref[...] = jnp.zeros((8, _B), jnp.int32)

    s = s_ref[...]
    nrm = jnp.sqrt(jnp.sum(s * s, axis=1, keepdims=True))
    sn = s / jnp.maximum(nrm, 1e-8)
    sn_ref[...] = jnp.concatenate(
        [sn, jnp.zeros((_T, _DP - _D), jnp.float32)], axis=1)
    sc = _mm(sn, qn_ref[...], ((1,), (1,)))
    g = jnp.max(sc.reshape(_GPT, _W, _B), axis=1)
    sub_g = jax.lax.broadcasted_iota(jnp.int32, (_GPT, 1), 0)
    gid = i * _GPT + sub_g
    g = jnp.where(gid < _NG, g, -jnp.inf)

    A = jnp.concatenate([cv_ref[...], g], axis=0)
    AI = jnp.concatenate(
        [ci_ref[...], jnp.broadcast_to(gid, (_GPT, _B))], axis=0)

    nv, ni = [], []
    for _ in range(_K):
        m = jnp.max(A, axis=0, keepdims=True)
        sel = jnp.min(jnp.where(A == m, AI, _BIG), axis=0, keepdims=True)
        nv.append(m)
        ni.append(sel)
        A = jnp.where(AI == sel, -jnp.inf, A)

    sub8 = jax.lax.broadcasted_iota(jnp.int32, (8, 1), 0)
    cv = jnp.full((8, _B), -jnp.inf, jnp.float32)
    ci = jnp.zeros((8, _B), jnp.int32)
    for j in range(_K):
        selj = sub8 == j
        cv = jnp.where(selj, nv[j], cv)
        ci = jnp.where(selj, ni[j], ci)
    cv_ref[...] = cv
    ci_ref[...] = ci

    @pl.when(i == _NT - 1)
    def _emit():
        sub_c = jax.lax.broadcasted_iota(jnp.int32, (_NC, 1), 0)
        jsel = sub_c // _W
        grp = jnp.zeros((_NC, _B), jnp.int32)
        for j in range(_K):
            grp = jnp.where(jsel == j, ni[j], grp)
        cand_ref[...] = grp * _W + sub_c % _W


def _knn_groups(x, w1, b1, g, bt, w2, b2, storage):
    const = lambda i: (0, 0)
    return pl.pallas_call(
        _knn_body,
        grid=(_NT,),
        in_specs=[
            pl.BlockSpec((_B, _D), const),
            pl.BlockSpec((_D, _D), const),
            pl.BlockSpec((1, _D), const),
            pl.BlockSpec((1, _D), const),
            pl.BlockSpec((1, _D), const),
            pl.BlockSpec((_D, _D), const),
            pl.BlockSpec((1, _D), const),
            pl.BlockSpec((_T, _D), lambda i: (i, 0)),
        ],
        out_specs=[
            pl.BlockSpec((_NC, _B), const),
            pl.BlockSpec((_T, _DP), lambda i: (i, 0)),
            pl.BlockSpec((_B, _D), const),
            pl.BlockSpec((_B, _D), const),
        ],
        out_shape=[jax.ShapeDtypeStruct((_NC, _B), jnp.int32),
                   jax.ShapeDtypeStruct((_MP, _DP), jnp.float32),
                   jax.ShapeDtypeStruct((_B, _D), jnp.float32),
                   jax.ShapeDtypeStruct((_B, _D), jnp.float32)],
        scratch_shapes=[
            pltpu.VMEM((_B, _D), jnp.float32),
            pltpu.VMEM((8, _B), jnp.float32),
            pltpu.VMEM((8, _B), jnp.int32),
        ],
        compiler_params=pltpu.CompilerParams(
            dimension_semantics=("arbitrary",)),
    )(x, w1, b1.reshape(1, _D), g.reshape(1, _D), bt.reshape(1, _D),
      w2, b2.reshape(1, _D), storage)


def _gather_rows(table, idx_flat, chunk, width, sc_tiling):
    n = idx_flat.shape[0]
    nw = 32
    bpw = n // nw
    nch = bpw // chunk
    mesh = plsc.VectorSubcoreMesh(core_axis_name="c", subcore_axis_name="s")

    @functools.partial(
        pl.kernel,
        mesh=mesh,
        out_type=jax.ShapeDtypeStruct((n, width), jnp.float32),
        scratch_types=[
            pltpu.VMEM((chunk,), jnp.int32),
            pltpu.VMEM((chunk, width), jnp.float32),
            pltpu.SemaphoreType.DMA,
        ],
        compiler_params=pltpu.CompilerParams(
            use_tc_tiling_on_sc=not sc_tiling),
    )
    def k(table_hbm, idx_hbm, out_hbm, idx_v, rows_v, sem):
        wid = lax.axis_index("s") * 2 + lax.axis_index("c")
        base = wid * bpw
        for ch in range(nch):
            off = base + ch * chunk
            pltpu.sync_copy(idx_hbm.at[pl.ds(off, chunk)], idx_v)
            pltpu.async_copy(table_hbm.at[idx_v], rows_v, sem).wait()
            pltpu.sync_copy(rows_v, out_hbm.at[pl.ds(off, chunk)])

    return k(table, idx_flat)


def _resolve_body(qn_ref, rows_ref, cand_ref, oi_ref):
    qn = qn_ref[...]
    scs = []
    for c in range(_NC):
        seg = rows_ref[c][:, :_D]
        scs.append(jnp.sum(seg * qn, axis=1, keepdims=True))
    S = jnp.concatenate(scs, axis=1)
    AI = cand_ref[...]
    ids = []
    for _ in range(_K):
        m = jnp.max(S, axis=1, keepdims=True)
        sel = jnp.min(jnp.where(S == m, AI, _BIG), axis=1, keepdims=True)
        ids.append(sel)
        S = jnp.where(AI == sel, -jnp.inf, S)
    lane5 = jax.lax.broadcasted_iota(jnp.int32, (1, _K), 1)
    oi = jnp.zeros((qn.shape[0], _K), jnp.int32)
    for j in range(_K):
        oi = jnp.where(lane5 == j, ids[j], oi)
    oi_ref[...] = oi


def _resolve(qn, rows, cand_q):
    nb = 4
    bb = _B // nb
    return pl.pallas_call(
        _resolve_body,
        grid=(nb,),
        in_specs=[
            pl.BlockSpec((bb, _D), lambda i: (i, 0)),
            pl.BlockSpec((_NC, bb, _DP), lambda i: (0, i, 0)),
            pl.BlockSpec((bb, _NC), lambda i: (i, 0)),
        ],
        out_specs=pl.BlockSpec((bb, _K), lambda i: (i, 0)),
        out_shape=jax.ShapeDtypeStruct((_B, _K), jnp.int32),
        compiler_params=pltpu.CompilerParams(
            dimension_semantics=("parallel",)),
    )(qn, rows, cand_q)


def _post_body(x_ref, eq_ref, r_ref, wq_ref, bq_ref, wk_ref, bk_ref,
               wv_ref, bv_ref, wo_ref, bo_ref, c1w_ref, c1b_ref,
               c2w_ref, c2b_ref, seg_ref, out_ref):
    eq = eq_ref[...]
    q = _mm(eq, wq_ref[...], ((1,), (1,))) + bq_ref[...]
    seg = seg_ref[...]
    scs, vs = [], []
    for j in range(_K):
        r = r_ref[:, j * _D:(j + 1) * _D]
        kj = _mm(r, wk_ref[...], ((1,), (1,))) + bk_ref[...]
        vj = _mm(r, wv_ref[...], ((1,), (1,))) + bv_ref[...]
        sj = _mm(q * kj, seg, ((1,), (1,)))
        scs.append(sj * 0.25)
        vs.append(vj)
    m = scs[0]
    for j in range(1, _K):
        m = jnp.maximum(m, scs[j])
    es = [jnp.exp(s - m) for s in scs]
    tot = es[0]
    for j in range(1, _K):
        tot = tot + es[j]
    ctx = jnp.zeros((_B, _D), jnp.float32)
    for j in range(_K):
        a = es[j] / tot
        ab = _mm(a, seg, ((1,), (0,)))
        ctx = ctx + ab * vs[j]
    comp = _mm(ctx, wo_ref[...], ((1,), (1,))) + bo_ref[...]
    h = _mm(comp, c1w_ref[...], ((1,), (1,))) + c1b_ref[...]
    h = _gelu_exact(h)
    ca1 = _mm(h, c2w_ref[...], ((1,), (1,))) + c2b_ref[...]
    out_ref[...] = x_ref[...] + 0.5 * ca1


def _post(x, eq, retrieved_flat, in_proj_w, in_proj_b, out_proj_w, out_proj_b,
          c1_W, c1_b, c2_W, c2_b):
    seg = np.zeros((_H, _D), np.float32)
    for h in range(_H):
        seg[h, h * 16:(h + 1) * 16] = 1.0
    seg = jnp.asarray(seg)
    wq, wk, wv = in_proj_w[:_D], in_proj_w[_D:2 * _D], in_proj_w[2 * _D:]
    bq, bk, bv = in_proj_b[:_D], in_proj_b[_D:2 * _D], in_proj_b[2 * _D:]
    return pl.pallas_call(
        _post_body,
        out_shape=jax.ShapeDtypeStruct((_B, _D), jnp.float32),
    )(x, eq, retrieved_flat, wq, bq.reshape(1, _D), wk, bk.reshape(1, _D),
      wv, bv.reshape(1, _D), out_proj_w, out_proj_b.reshape(1, _D),
      c1_W, c1_b.reshape(1, 2 * _D), c2_W, c2_b.reshape(1, _D), seg)


def kernel(x, k_W1, k_b1, k_gamma, k_beta, k_W2, k_b2, storage, memory_values,
           in_proj_w, in_proj_b, out_proj_w, out_proj_b, c1_W, c1_b, c2_W,
           c2_b):
    cand_t, sn_full, eq, qn = _knn_groups(
        x, k_W1, k_b1, k_gamma, k_beta, k_W2, k_b2, storage)
    crows = _gather_rows(sn_full, cand_t.reshape(_NC * _B),
                         chunk=640, width=_DP, sc_tiling=False)
    top = _resolve(qn, crows.reshape(_NC, _B, _DP),
                   jnp.transpose(cand_t))
    idx_flat = top.reshape(_B * _K)
    retrieved = _gather_rows(memory_values, idx_flat,
                             chunk=160, width=_D, sc_tiling=True)
    retrieved_flat = retrieved.reshape(_B, _K * _D)
    return _post(x, eq, retrieved_flat, in_proj_w, in_proj_b,
                 out_proj_w, out_proj_b, c1_W, c1_b, c2_W, c2_b)


# speedup vs baseline: 4.4302x; 1.0024x over previous
import functools

import jax
import jax.numpy as jnp
import numpy as np
from jax import lax
from jax.experimental import pallas as pl
from jax.experimental.pallas import tpu as pltpu
from jax.experimental.pallas import tpu_sc as plsc

_B = 1024
_D = 64
_DP = 128
_M = 100000
_H = 4
_K = 5
_T = 2048
_NT = (_M + _T - 1) // _T
_MP = _NT * _T
_W = 16
_GPT = _T // _W
_NG = _M // _W
_NC = _K * _W
_BIG = 2 ** 30


def _gelu_exact(h):
    return 0.5 * h * (1.0 + lax.erf(h * np.float32(0.7071067811865476)))


def _mm(a, b, dims):
    return jax.lax.dot_general(a, b, (dims, ((), ())),
                               preferred_element_type=jnp.float32)


def _knn_body(x_ref, w1_ref, b1_ref, g_ref, bt_ref, w2_ref, b2_ref,
              s_ref, cand_ref, sn_ref, eq_ref, qn_out_ref, qn_ref,
              cv_ref, ci_ref):
    i = pl.program_id(0)

    @pl.when(i == 0)
    def _init():
        x = x_ref[...]
        h = _mm(x, w1_ref[...], ((1,), (1,))) + b1_ref[...]
        mu = jnp.mean(h, axis=-1, keepdims=True)
        var = jnp.mean((h - mu) ** 2, axis=-1, keepdims=True)
        h = (h - mu) / jnp.sqrt(var + 1e-5) * g_ref[...] + bt_ref[...]
        h = _gelu_exact(h)
        eq = _mm(h, w2_ref[...], ((1,), (1,))) + b2_ref[...]
        eq_ref[...] = eq
        qnrm = jnp.sqrt(jnp.sum(eq * eq, axis=-1, keepdims=True))
        qn = eq / jnp.maximum(qnrm, 1e-8)
        qn_ref[...] = qn
        qn_out_ref[...] = qn
        cv_ref[...] = jnp.full((8, _B), -jnp.inf, jnp.float32)
        ci_ref[...] = jnp.zeros((8, _B), jnp.int32)

    s = s_ref[...]
    nrm = jnp.sqrt(jnp.sum(s * s, axis=1, keepdims=True))
    sn = s * (1.0 / jnp.maximum(nrm, 1e-8))
    sn_ref[...] = jnp.concatenate(
        [sn, jnp.zeros((_T, _DP - _D), jnp.float32)], axis=1)
    sc = _mm(sn, qn_ref[...], ((1,), (1,)))
    g = jnp.max(sc.reshape(_GPT, _W, _B), axis=1)
    sub_g = jax.lax.broadcasted_iota(jnp.int32, (_GPT, 1), 0)
    gid = i * _GPT + sub_g
    g = jnp.where(gid < _NG, g, -jnp.inf)

    A = jnp.concatenate([cv_ref[...], g], axis=0)
    AI = jnp.concatenate(
        [ci_ref[...], jnp.broadcast_to(gid, (_GPT, _B))], axis=0)

    nv, ni = [], []
    for _ in range(_K):
        m = jnp.max(A, axis=0, keepdims=True)
        sel = jnp.min(jnp.where(A == m, AI, _BIG), axis=0, keepdims=True)
        nv.append(m)
        ni.append(sel)
        A = jnp.where(AI == sel, -jnp.inf, A)

    sub8 = jax.lax.broadcasted_iota(jnp.int32, (8, 1), 0)
    cv = jnp.full((8, _B), -jnp.inf, jnp.float32)
    ci = jnp.zeros((8, _B), jnp.int32)
    for j in range(_K):
        selj = sub8 == j
        cv = jnp.where(selj, nv[j], cv)
        ci = jnp.where(selj, ni[j], ci)
    cv_ref[...] = cv
    ci_ref[...] = ci

    @pl.when(i == _NT - 1)
    def _emit():
        sub_c = jax.lax.broadcasted_iota(jnp.int32, (_NC, 1), 0)
        jsel = sub_c // _W
        grp = jnp.zeros((_NC, _B), jnp.int32)
        for j in range(_K):
            grp = jnp.where(jsel == j, ni[j], grp)
        cand_ref[...] = grp * _W + sub_c % _W


def _knn_groups(x, w1, b1, g, bt, w2, b2, storage):
    const = lambda i: (0, 0)
    return pl.pallas_call(
        _knn_body,
        grid=(_NT,),
        in_specs=[
            pl.BlockSpec((_B, _D), const),
            pl.BlockSpec((_D, _D), const),
            pl.BlockSpec((1, _D), const),
            pl.BlockSpec((1, _D), const),
            pl.BlockSpec((1, _D), const),
            pl.BlockSpec((_D, _D), const),
            pl.BlockSpec((1, _D), const),
            pl.BlockSpec((_T, _D), lambda i: (i, 0)),
        ],
        out_specs=[
            pl.BlockSpec((_NC, _B), const),
            pl.BlockSpec((_T, _DP), lambda i: (i, 0)),
            pl.BlockSpec((_B, _D), const),
            pl.BlockSpec((_B, _D), const),
        ],
        out_shape=[jax.ShapeDtypeStruct((_NC, _B), jnp.int32),
                   jax.ShapeDtypeStruct((_MP, _DP), jnp.float32),
                   jax.ShapeDtypeStruct((_B, _D), jnp.float32),
                   jax.ShapeDtypeStruct((_B, _D), jnp.float32)],
        scratch_shapes=[
            pltpu.VMEM((_B, _D), jnp.float32),
            pltpu.VMEM((8, _B), jnp.float32),
            pltpu.VMEM((8, _B), jnp.int32),
        ],
        compiler_params=pltpu.CompilerParams(
            dimension_semantics=("arbitrary",)),
    )(x, w1, b1.reshape(1, _D), g.reshape(1, _D), bt.reshape(1, _D),
      w2, b2.reshape(1, _D), storage)


def _gather_rows(table, idx_flat, chunk, width, sc_tiling):
    n = idx_flat.shape[0]
    nw = 32
    bpw = n // nw
    nch = bpw // chunk
    mesh = plsc.VectorSubcoreMesh(core_axis_name="c", subcore_axis_name="s")

    @functools.partial(
        pl.kernel,
        mesh=mesh,
        out_type=jax.ShapeDtypeStruct((n, width), jnp.float32),
        scratch_types=[
            pltpu.VMEM((chunk,), jnp.int32),
            pltpu.VMEM((chunk, width), jnp.float32),
            pltpu.SemaphoreType.DMA,
        ],
        compiler_params=pltpu.CompilerParams(
            use_tc_tiling_on_sc=not sc_tiling),
    )
    def k(table_hbm, idx_hbm, out_hbm, idx_v, rows_v, sem):
        wid = lax.axis_index("s") * 2 + lax.axis_index("c")
        base = wid * bpw
        for ch in range(nch):
            off = base + ch * chunk
            pltpu.sync_copy(idx_hbm.at[pl.ds(off, chunk)], idx_v)
            pltpu.async_copy(table_hbm.at[idx_v], rows_v, sem).wait()
            pltpu.sync_copy(rows_v, out_hbm.at[pl.ds(off, chunk)])

    return k(table, idx_flat)


def _resolve_body(qn_ref, rows_ref, cand_ref, oi_ref):
    qn = qn_ref[...]
    scs = []
    for c in range(_NC):
        seg = rows_ref[c][:, :_D]
        scs.append(jnp.sum(seg * qn, axis=1, keepdims=True))
    S = jnp.concatenate(scs, axis=1)
    AI = jnp.transpose(cand_ref[...])
    ids = []
    for _ in range(_K):
        m = jnp.max(S, axis=1, keepdims=True)
        sel = jnp.min(jnp.where(S == m, AI, _BIG), axis=1, keepdims=True)
        ids.append(sel)
        S = jnp.where(AI == sel, -jnp.inf, S)
    lane5 = jax.lax.broadcasted_iota(jnp.int32, (1, _K), 1)
    oi = jnp.zeros((qn.shape[0], _K), jnp.int32)
    for j in range(_K):
        oi = jnp.where(lane5 == j, ids[j], oi)
    oi_ref[...] = oi


def _resolve(qn, rows, cand_q):
    nb = 4
    bb = _B // nb
    return pl.pallas_call(
        _resolve_body,
        grid=(nb,),
        in_specs=[
            pl.BlockSpec((bb, _D), lambda i: (i, 0)),
            pl.BlockSpec((_NC, bb, _DP), lambda i: (0, i, 0)),
            pl.BlockSpec((_NC, bb), lambda i: (0, i)),
        ],
        out_specs=pl.BlockSpec((bb, _K), lambda i: (i, 0)),
        out_shape=jax.ShapeDtypeStruct((_B, _K), jnp.int32),
        compiler_params=pltpu.CompilerParams(
            dimension_semantics=("parallel",)),
    )(qn, rows, cand_q)


def _post_body(x_ref, eq_ref, r_ref, wq_ref, bq_ref, wk_ref, bk_ref,
               wv_ref, bv_ref, wo_ref, bo_ref, c1w_ref, c1b_ref,
               c2w_ref, c2b_ref, seg_ref, out_ref):
    eq = eq_ref[...]
    q = _mm(eq, wq_ref[...], ((1,), (1,))) + bq_ref[...]
    seg = seg_ref[...]
    scs, vs = [], []
    for j in range(_K):
        r = r_ref[:, j * _D:(j + 1) * _D]
        kj = _mm(r, wk_ref[...], ((1,), (1,))) + bk_ref[...]
        vj = _mm(r, wv_ref[...], ((1,), (1,))) + bv_ref[...]
        sj = _mm(q * kj, seg, ((1,), (1,)))
        scs.append(sj * 0.25)
        vs.append(vj)
    m = scs[0]
    for j in range(1, _K):
        m = jnp.maximum(m, scs[j])
    es = [jnp.exp(s - m) for s in scs]
    tot = es[0]
    for j in range(1, _K):
        tot = tot + es[j]
    ctx = jnp.zeros((_B, _D), jnp.float32)
    for j in range(_K):
        a = es[j] / tot
        ab = _mm(a, seg, ((1,), (0,)))
        ctx = ctx + ab * vs[j]
    comp = _mm(ctx, wo_ref[...], ((1,), (1,))) + bo_ref[...]
    h = _mm(comp, c1w_ref[...], ((1,), (1,))) + c1b_ref[...]
    h = _gelu_exact(h)
    ca1 = _mm(h, c2w_ref[...], ((1,), (1,))) + c2b_ref[...]
    out_ref[...] = x_ref[...] + 0.5 * ca1


def _post(x, eq, retrieved_flat, in_proj_w, in_proj_b, out_proj_w, out_proj_b,
          c1_W, c1_b, c2_W, c2_b):
    seg = np.zeros((_H, _D), np.float32)
    for h in range(_H):
        seg[h, h * 16:(h + 1) * 16] = 1.0
    seg = jnp.asarray(seg)
    wq, wk, wv = in_proj_w[:_D], in_proj_w[_D:2 * _D], in_proj_w[2 * _D:]
    bq, bk, bv = in_proj_b[:_D], in_proj_b[_D:2 * _D], in_proj_b[2 * _D:]
    return pl.pallas_call(
        _post_body,
        out_shape=jax.ShapeDtypeStruct((_B, _D), jnp.float32),
    )(x, eq, retrieved_flat, wq, bq.reshape(1, _D), wk, bk.reshape(1, _D),
      wv, bv.reshape(1, _D), out_proj_w, out_proj_b.reshape(1, _D),
      c1_W, c1_b.reshape(1, 2 * _D), c2_W, c2_b.reshape(1, _D), seg)


def kernel(x, k_W1, k_b1, k_gamma, k_beta, k_W2, k_b2, storage, memory_values,
           in_proj_w, in_proj_b, out_proj_w, out_proj_b, c1_W, c1_b, c2_W,
           c2_b):
    cand_t, sn_full, eq, qn = _knn_groups(
        x, k_W1, k_b1, k_gamma, k_beta, k_W2, k_b2, storage)
    crows = _gather_rows(sn_full, cand_t.reshape(_NC * _B),
                         chunk=640, width=_DP, sc_tiling=False)
    top = _resolve(qn, crows.reshape(_NC, _B, _DP), cand_t)
    idx_flat = top.reshape(_B * _K)
    retrieved = _gather_rows(memory_values, idx_flat,
                             chunk=160, width=_D, sc_tiling=True)
    retrieved_flat = retrieved.reshape(_B, _K * _D)
    return _post(x, eq, retrieved_flat, in_proj_w, in_proj_b,
                 out_proj_w, out_proj_b, c1_W, c1_b, c2_W, c2_b)
